# Initial kernel scaffold; baseline (speedup 1.0000x reference)
import functools

import jax
import jax.numpy as jnp
from jax import lax
from jax.experimental import pallas as pl
from jax.experimental.pallas import tpu as pltpu
from jax.experimental.pallas import tpu_sc as plsc

N = 10000
E = 320000
D = 128
H = 128

NC = 2
NS = 16
EW = E // (NC * NS)
K = 80
NCHUNK = EW // K
ROWS_PER_TILE = N // NS
ZR = 25
CZR = 125


def _sc_body(x_hbm, src_hbm, dst_hbm, agg_out, cnt_out,
             zrow, czero, ones16, srcbuf, dstbuf, rows, agg_s, cnt_s, sem):
    c = lax.axis_index("c")
    s = lax.axis_index("s")
    wid = c * NS + s

    zero16 = jnp.zeros((16,), jnp.float32)
    one16 = jnp.ones((16,), jnp.float32)

    def fill_zrow(i, _):
        for j in range(D // 16):
            zrow[i, pl.ds(j * 16, 16)] = zero16
        return 0
    lax.fori_loop(0, ZR, fill_zrow, 0)

    def fill_czero(i, _):
        czero[i, :] = zero16
        return 0
    lax.fori_loop(0, CZR, fill_czero, 0)

    def fill_ones(i, _):
        ones16[i, :] = one16
        return 0
    lax.fori_loop(0, K, fill_ones, 0)

    r0 = s * ROWS_PER_TILE

    def zero_agg(i, _):
        pltpu.sync_copy(zrow, agg_s.at[pl.ds(r0 + i * ZR, ZR)])
        return 0
    lax.fori_loop(0, ROWS_PER_TILE // ZR, zero_agg, 0)

    def zero_cnt(i, _):
        pltpu.sync_copy(czero, cnt_s.at[pl.ds(r0 + i * CZR, CZR)])
        return 0
    lax.fori_loop(0, ROWS_PER_TILE // CZR, zero_cnt, 0)

    plsc.subcore_barrier()

    base = wid * EW

    def chunk(k, _):
        off = base + k * K
        pltpu.sync_copy(src_hbm.at[pl.ds(off, K)], srcbuf)
        pltpu.sync_copy(dst_hbm.at[pl.ds(off, K)], dstbuf)
        pltpu.async_copy(x_hbm.at[srcbuf], rows, sem).wait()
        pltpu.sync_copy(rows, agg_s.at[dstbuf], add=True)
        pltpu.sync_copy(ones16, cnt_s.at[dstbuf], add=True)
        return 0
    lax.fori_loop(0, NCHUNK, chunk, 0)

    plsc.subcore_barrier()

    pltpu.sync_copy(agg_s.at[pl.ds(r0, ROWS_PER_TILE)],
                    agg_out.at[c, pl.ds(r0, ROWS_PER_TILE)])
    pltpu.sync_copy(cnt_s.at[pl.ds(r0, ROWS_PER_TILE)],
                    cnt_out.at[c, pl.ds(r0, ROWS_PER_TILE)])


_sc_agg = functools.partial(
    pl.kernel,
    mesh=plsc.VectorSubcoreMesh(core_axis_name="c", subcore_axis_name="s"),
    out_type=[jax.ShapeDtypeStruct((NC, N, D), jnp.float32),
              jax.ShapeDtypeStruct((NC, N, 16), jnp.float32)],
    scratch_types=[
        pltpu.VMEM((ZR, D), jnp.float32),
        pltpu.VMEM((CZR, 16), jnp.float32),
        pltpu.VMEM((K, 16), jnp.float32),
        pltpu.VMEM((K,), jnp.int32),
        pltpu.VMEM((K,), jnp.int32),
        pltpu.VMEM((K, D), jnp.float32),
        pltpu.VMEM_SHARED((N, D), jnp.float32),
        pltpu.VMEM_SHARED((N, 16), jnp.float32),
        pltpu.SemaphoreType.DMA,
    ],
)(_sc_body)


def _tc_body(agg_ref, cnt_ref, x_ref, wl_ref, bl_ref, wr_ref, wh_ref, bh_ref,
             out_ref):
    p = agg_ref[0] + agg_ref[1]
    cnt = cnt_ref[0] + cnt_ref[1]
    inv = 1.0 / jnp.maximum(cnt[:, 0:1], 1.0)
    aggm = p * inv
    h = jnp.dot(aggm, wl_ref[...], preferred_element_type=jnp.float32)
    h = h + bl_ref[...]
    h = h + jnp.dot(x_ref[...], wr_ref[...], preferred_element_type=jnp.float32)
    h = jnp.maximum(h, 0.0)
    logits = jnp.dot(h, wh_ref[...], preferred_element_type=jnp.float32)
    logits = logits + bh_ref[...]

    def lsm(z):
        m = jnp.max(z, axis=1, keepdims=True)
        zz = z - m
        return zz - jnp.log(jnp.sum(jnp.exp(zz), axis=1, keepdims=True))

    out_ref[:, 0:64] = lsm(logits[:, 0:64])
    out_ref[:, 64:128] = lsm(logits[:, 64:128])


def _tc_head(agg_p, cnt_p, x, W_l, b_l, W_r, Wh, bh):
    B = 2000
    grid = (N // B,)
    return pl.pallas_call(
        _tc_body,
        grid=grid,
        in_specs=[
            pl.BlockSpec((NC, B, D), lambda i: (0, i, 0)),
            pl.BlockSpec((NC, B, 16), lambda i: (0, i, 0)),
            pl.BlockSpec((B, D), lambda i: (i, 0)),
            pl.BlockSpec((D, H), lambda i: (0, 0)),
            pl.BlockSpec((1, H), lambda i: (0, 0)),
            pl.BlockSpec((D, H), lambda i: (0, 0)),
            pl.BlockSpec((H, 128), lambda i: (0, 0)),
            pl.BlockSpec((1, 128), lambda i: (0, 0)),
        ],
        out_specs=pl.BlockSpec((B, 128), lambda i: (i, 0)),
        out_shape=jax.ShapeDtypeStruct((N, 128), jnp.float32),
    )(agg_p, cnt_p, x, W_l, b_l, W_r, Wh, bh)


def kernel(x, edge_index, W_l, b_l, W_r, W_p, b_p, W_s, b_s):
    src = edge_index[0]
    dst = edge_index[1]
    agg_p, cnt_p = _sc_agg(x, src, dst)

    Wh = jnp.zeros((H, 128), jnp.float32)
    Wh = Wh.at[:, 0:7].set(W_p).at[:, 64:70].set(W_s)
    bh = jnp.full((128,), -1e30, jnp.float32)
    bh = bh.at[0:7].set(b_p).at[64:70].set(b_s)

    out = _tc_head(agg_p, cnt_p, x, W_l, b_l.reshape(1, H), W_r,
                   Wh, bh.reshape(1, 128))
    return (out[:, 0:7], out[:, 64:70])


# speedup vs baseline: 5.9247x; 5.9247x over previous
import functools

import jax
import jax.numpy as jnp
from jax import lax
from jax.experimental import pallas as pl
from jax.experimental.pallas import tpu as pltpu
from jax.experimental.pallas import tpu_sc as plsc

N = 10000
E = 320000
D = 128
H = 128

NC = 2
NS = 16
EW = E // (NC * NS)
K = 80
NCHUNK = EW // K
NP = 10240
ROWS_PER_TILE = NP // NS
ZR = 32


def _sc_body(x_hbm, src_hbm, dst_hbm, agg_out, cnt_out,
             zrow, czero, ones1, srcbuf, dstbuf, rows, agg_s, cnt_s, sem):
    c = lax.axis_index("c")
    s = lax.axis_index("s")
    wid = c * NS + s

    zero16 = jnp.zeros((16,), jnp.float32)
    one16 = jnp.ones((16,), jnp.float32)

    def fill_zrow(i, _):
        for j in range(D // 16):
            zrow[i, pl.ds(j * 16, 16)] = zero16
        return 0
    lax.fori_loop(0, ZR, fill_zrow, 0)

    def fill_czero(i, _):
        czero[pl.ds(i * 16, 16)] = zero16
        return 0
    lax.fori_loop(0, ROWS_PER_TILE // 16, fill_czero, 0)

    def fill_ones(i, _):
        ones1[pl.ds(i * 16, 16)] = one16
        return 0
    lax.fori_loop(0, K // 16, fill_ones, 0)

    r0 = s * ROWS_PER_TILE

    def zero_agg(i, _):
        pltpu.sync_copy(zrow, agg_s.at[pl.ds(r0 + i * ZR, ZR)])
        return 0
    lax.fori_loop(0, ROWS_PER_TILE // ZR, zero_agg, 0)

    pltpu.sync_copy(czero, cnt_s.at[pl.ds(r0, ROWS_PER_TILE)])

    plsc.subcore_barrier()

    base = wid * EW

    def chunk(k, _):
        off = base + k * K
        pltpu.sync_copy(src_hbm.at[pl.ds(off, K)], srcbuf)
        pltpu.sync_copy(dst_hbm.at[pl.ds(off, K)], dstbuf)
        pltpu.async_copy(x_hbm.at[srcbuf], rows, sem).wait()
        pltpu.sync_copy(rows, agg_s.at[dstbuf], add=True)
        pltpu.sync_copy(ones1, cnt_s.at[dstbuf], add=True)
        return 0
    lax.fori_loop(0, NCHUNK, chunk, 0)

    plsc.subcore_barrier()

    pltpu.sync_copy(agg_s.at[pl.ds(r0, ROWS_PER_TILE)],
                    agg_out.at[c, pl.ds(r0, ROWS_PER_TILE)])
    pltpu.sync_copy(cnt_s.at[pl.ds(r0, ROWS_PER_TILE)],
                    cnt_out.at[pl.ds(c * NP + r0, ROWS_PER_TILE)])


@functools.lru_cache(maxsize=1)
def _sc_agg():
    return pl.kernel(
        _sc_body,
        mesh=plsc.VectorSubcoreMesh(core_axis_name="c", subcore_axis_name="s",
                                    num_cores=NC, num_subcores=NS),
        out_type=[jax.ShapeDtypeStruct((NC, NP, D), jnp.float32),
                  jax.ShapeDtypeStruct((NC * NP,), jnp.float32)],
        scratch_types=[
            pltpu.VMEM((ZR, D), jnp.float32),
            pltpu.VMEM((ROWS_PER_TILE,), jnp.float32),
            pltpu.VMEM((K,), jnp.float32),
            pltpu.VMEM((K,), jnp.int32),
            pltpu.VMEM((K,), jnp.int32),
            pltpu.VMEM((K, D), jnp.float32),
            pltpu.VMEM_SHARED((NP, D), jnp.float32),
            pltpu.VMEM_SHARED((NP,), jnp.float32),
            pltpu.SemaphoreType.DMA,
        ],
    )


def _tc_body(agg_ref, cnt_ref, x_ref, wl_ref, bl_ref, wr_ref, wh_ref, bh_ref,
             out_ref):
    p = agg_ref[0] + agg_ref[1]
    cnt = cnt_ref[0] + cnt_ref[1]
    inv = 1.0 / jnp.maximum(cnt, 1.0)
    aggm = p * inv
    h = jnp.dot(aggm, wl_ref[...], preferred_element_type=jnp.float32)
    h = h + bl_ref[...]
    h = h + jnp.dot(x_ref[...], wr_ref[...], preferred_element_type=jnp.float32)
    h = jnp.maximum(h, 0.0)
    logits = jnp.dot(h, wh_ref[...], preferred_element_type=jnp.float32)
    logits = logits + bh_ref[...]

    def lsm(z):
        m = jnp.max(z, axis=1, keepdims=True)
        zz = z - m
        return zz - jnp.log(jnp.sum(jnp.exp(zz), axis=1, keepdims=True))

    out_ref[:, 0:64] = lsm(logits[:, 0:64])
    out_ref[:, 64:128] = lsm(logits[:, 64:128])


def _tc_head(agg_p, cnt_p, x, W_l, b_l, W_r, Wh, bh):
    B = 2000
    grid = (N // B,)
    return pl.pallas_call(
        _tc_body,
        grid=grid,
        in_specs=[
            pl.BlockSpec((NC, B, D), lambda i: (0, i, 0)),
            pl.BlockSpec((NC, B, 1), lambda i: (0, i, 0)),
            pl.BlockSpec((B, D), lambda i: (i, 0)),
            pl.BlockSpec((D, H), lambda i: (0, 0)),
            pl.BlockSpec((1, H), lambda i: (0, 0)),
            pl.BlockSpec((D, H), lambda i: (0, 0)),
            pl.BlockSpec((H, 128), lambda i: (0, 0)),
            pl.BlockSpec((1, 128), lambda i: (0, 0)),
        ],
        out_specs=pl.BlockSpec((B, 128), lambda i: (i, 0)),
        out_shape=jax.ShapeDtypeStruct((N, 128), jnp.float32),
    )(agg_p, cnt_p, x, W_l, b_l, W_r, Wh, bh)


def kernel(x, edge_index, W_l, b_l, W_r, W_p, b_p, W_s, b_s):
    src = edge_index[0]
    dst = edge_index[1]
    agg_p, cnt_p = _sc_agg()(x, src, dst)
    cnt_p = cnt_p.reshape(NC, NP, 1)

    Wh = jnp.zeros((H, 128), jnp.float32)
    Wh = Wh.at[:, 0:7].set(W_p).at[:, 64:70].set(W_s)
    bh = jnp.full((128,), -1e30, jnp.float32)
    bh = bh.at[0:7].set(b_p).at[64:70].set(b_s)

    out = _tc_head(agg_p, cnt_p, x, W_l, b_l.reshape(1, H), W_r,
                   Wh, bh.reshape(1, 128))
    return (out[:, 0:7], out[:, 64:70])


# speedup vs baseline: 11.3302x; 1.9124x over previous
import functools

import jax
import jax.numpy as jnp
from jax import lax
from jax.experimental import pallas as pl
from jax.experimental.pallas import tpu as pltpu
from jax.experimental.pallas import tpu_sc as plsc

N = 10000
E = 320000
D = 128
H = 128

NC = 2
NS = 16
EW = E // (NC * NS)
K = 40
NCHUNK = EW // K
NP = 10240
ROWS_PER_TILE = NP // NS
ZR = 32
NBUF = 5
NW = NC * NS


def _sc_body(x_hbm, src_hbm, dst_hbm, agg_out, cnt_out,
             zrow, czero, ones1, rows, agg_s, cnt_s, *rest):
    srcb = rest[0:NBUF]
    didxb = rest[NBUF:2 * NBUF]
    isem = rest[2 * NBUF:3 * NBUF]
    gsem = rest[3 * NBUF:4 * NBUF]
    ssem = rest[4 * NBUF:5 * NBUF]
    zsem = rest[5 * NBUF]
    c = lax.axis_index("c")
    s = lax.axis_index("s")
    wid = c * NS + s

    zero16 = jnp.zeros((16,), jnp.float32)
    one16 = jnp.ones((16,), jnp.float32)

    def fill_zrow(i, _):
        for j in range(D // 16):
            zrow[i, pl.ds(j * 16, 16)] = zero16
        return 0
    lax.fori_loop(0, ZR, fill_zrow, 0)

    def fill_czero(i, _):
        czero[pl.ds(i * 16, 16)] = zero16
        return 0
    lax.fori_loop(0, ROWS_PER_TILE // 16, fill_czero, 0)

    for off in sorted(set(list(range(0, K - 15, 16)) + [K - 16])):
        ones1[pl.ds(off, 16)] = one16

    r0 = s * ROWS_PER_TILE
    base = wid * EW
    d_cz = pltpu.async_copy(czero, cnt_s.at[pl.ds(r0, ROWS_PER_TILE)], zsem)

    def zero_agg(i, _):
        pltpu.async_copy(zrow, agg_s.at[pl.ds(r0 + i * ZR, ZR)], zsem)
        return 0
    lax.fori_loop(0, ROWS_PER_TILE // ZR, zero_agg, 0)

    def zero_agg_wait(i, _):
        pltpu.make_async_copy(zrow, agg_s.at[pl.ds(r0, ZR)], zsem).wait()
        return 0
    lax.fori_loop(0, ROWS_PER_TILE // ZR, zero_agg_wait, 0)
    d_cz.wait()

    plsc.subcore_barrier()

    def load_idx(j, b):
        pltpu.async_copy(src_hbm.at[pl.ds(base + j * K, K)], srcb[b], isem[b])
        pltpu.async_copy(dst_hbm.at[pl.ds(base + j * K, K)], didxb[b],
                         isem[b])

    def wait_idx(b):
        pltpu.make_async_copy(src_hbm.at[pl.ds(base, K)], srcb[b],
                              isem[b]).wait()
        pltpu.make_async_copy(dst_hbm.at[pl.ds(base, K)], didxb[b],
                              isem[b]).wait()

    def issue_gather(b):
        pltpu.async_copy(x_hbm.at[srcb[b]], rows.at[b], gsem[b])

    def wait_gather(b):
        pltpu.make_async_copy(x_hbm.at[srcb[b]], rows.at[b], gsem[b]).wait()

    def issue_scatter(b):
        pltpu.async_copy(rows.at[b], agg_s.at[didxb[b]], ssem[b], add=True)
        pltpu.async_copy(ones1, cnt_s.at[didxb[b]], ssem[b], add=True)

    def wait_scatter(b):
        pltpu.make_async_copy(rows.at[b], agg_s.at[didxb[b]], ssem[b]).wait()
        pltpu.make_async_copy(ones1, cnt_s.at[didxb[b]], ssem[b]).wait()

    load_idx(0, 0)
    load_idx(1, 1)
    wait_idx(0)
    issue_gather(0)

    def outer(g, _):
        for b in range(NBUF):
            i = g * NBUF + b
            b1 = (b + 1) % NBUF
            b2 = (b + 2) % NBUF
            bp = (b - 1) % NBUF

            @pl.when(i + 2 < NCHUNK)
            def _load():
                load_idx(i + 2, b2)

            @pl.when(i + 1 < NCHUNK)
            def _gather():
                wait_idx(b1)
                issue_gather(b1)

            wait_gather(b)

            @pl.when(i >= 1)
            def _drain():
                wait_scatter(bp)

            issue_scatter(b)
        return 0
    lax.fori_loop(0, NCHUNK // NBUF, outer, 0)

    wait_scatter((NCHUNK - 1) % NBUF)

    plsc.subcore_barrier()

    pltpu.sync_copy(agg_s.at[pl.ds(r0, ROWS_PER_TILE)],
                    agg_out.at[c, pl.ds(r0, ROWS_PER_TILE)])
    pltpu.sync_copy(cnt_s.at[pl.ds(r0, ROWS_PER_TILE)],
                    cnt_out.at[pl.ds(c * NP + r0, ROWS_PER_TILE)])


@functools.lru_cache(maxsize=1)
def _sc_agg():
    return pl.kernel(
        _sc_body,
        mesh=plsc.VectorSubcoreMesh(core_axis_name="c", subcore_axis_name="s",
                                    num_cores=NC, num_subcores=NS),
        out_type=[jax.ShapeDtypeStruct((NC, NP, D), jnp.float32),
                  jax.ShapeDtypeStruct((NC * NP,), jnp.float32)],
        scratch_types=[
            pltpu.VMEM((ZR, D), jnp.float32),
            pltpu.VMEM((ROWS_PER_TILE,), jnp.float32),
            pltpu.VMEM((K,), jnp.float32),
            pltpu.VMEM((NBUF, K, D), jnp.float32),
            pltpu.VMEM_SHARED((NP, D), jnp.float32),
            pltpu.VMEM_SHARED((NP,), jnp.float32),
        ] + [pltpu.VMEM((K,), jnp.int32)] * (2 * NBUF)
          + [pltpu.SemaphoreType.DMA] * (3 * NBUF + 1),
    )


def _tc_body(agg_ref, cnt_ref, x_ref, wl_ref, bl_ref, wr_ref, wh_ref, bh_ref,
             out_ref):
    p = agg_ref[0] + agg_ref[1]
    cnt = cnt_ref[0] + cnt_ref[1]
    inv = 1.0 / jnp.maximum(cnt, 1.0)
    aggm = p * inv
    h = jnp.dot(aggm, wl_ref[...], preferred_element_type=jnp.float32)
    h = h + bl_ref[...]
    h = h + jnp.dot(x_ref[...], wr_ref[...], preferred_element_type=jnp.float32)
    h = jnp.maximum(h, 0.0)
    logits = jnp.dot(h, wh_ref[...], preferred_element_type=jnp.float32)
    logits = logits + bh_ref[...]

    def lsm(z):
        m = jnp.max(z, axis=1, keepdims=True)
        zz = z - m
        return zz - jnp.log(jnp.sum(jnp.exp(zz), axis=1, keepdims=True))

    out_ref[:, 0:64] = lsm(logits[:, 0:64])
    out_ref[:, 64:128] = lsm(logits[:, 64:128])


def _tc_head(agg_p, cnt_p, x, W_l, b_l, W_r, Wh, bh):
    B = 2000
    grid = (N // B,)
    return pl.pallas_call(
        _tc_body,
        grid=grid,
        in_specs=[
            pl.BlockSpec((NC, B, D), lambda i: (0, i, 0)),
            pl.BlockSpec((NC, B, 1), lambda i: (0, i, 0)),
            pl.BlockSpec((B, D), lambda i: (i, 0)),
            pl.BlockSpec((D, H), lambda i: (0, 0)),
            pl.BlockSpec((1, H), lambda i: (0, 0)),
            pl.BlockSpec((D, H), lambda i: (0, 0)),
            pl.BlockSpec((H, 128), lambda i: (0, 0)),
            pl.BlockSpec((1, 128), lambda i: (0, 0)),
        ],
        out_specs=pl.BlockSpec((B, 128), lambda i: (i, 0)),
        out_shape=jax.ShapeDtypeStruct((N, 128), jnp.float32),
    )(agg_p, cnt_p, x, W_l, b_l, W_r, Wh, bh)


def kernel(x, edge_index, W_l, b_l, W_r, W_p, b_p, W_s, b_s):
    src = edge_index[0]
    dst = edge_index[1]
    agg_p, cnt_p = _sc_agg()(x, src, dst)
    cnt_p = cnt_p.reshape(NC, NP, 1)

    Wh = jnp.zeros((H, 128), jnp.float32)
    Wh = Wh.at[:, 0:7].set(W_p).at[:, 64:70].set(W_s)
    bh = jnp.full((128,), -1e30, jnp.float32)
    bh = bh.at[0:7].set(b_p).at[64:70].set(b_s)

    out = _tc_head(agg_p, cnt_p, x, W_l, b_l.reshape(1, H), W_r,
                   Wh, bh.reshape(1, 128))
    return (out[:, 0:7], out[:, 64:70])


# speedup vs baseline: 13.8543x; 1.2228x over previous
import functools

import jax
import jax.numpy as jnp
from jax import lax
from jax.experimental import pallas as pl
from jax.experimental.pallas import tpu as pltpu
from jax.experimental.pallas import tpu_sc as plsc

N = 10000
E = 320000
D = 128
H = 128

NC = 2
NS = 16
EW = E // (NC * NS)
K = 40
NCHUNK = EW // K
NP = 10240
ROWS_PER_TILE = NP // NS
ZR = 32
NBUF = 5
NW = NC * NS


def _sc_body(x_hbm, src_hbm, dst_hbm, agg_out, cnt_out,
             zrow, czero, ones1, rows, agg_s, cnt_s, *rest):
    srcb = rest[0:NBUF]
    didxb = rest[NBUF:2 * NBUF]
    isem = rest[2 * NBUF:3 * NBUF]
    gsem = rest[3 * NBUF:4 * NBUF]
    ssem = rest[4 * NBUF:5 * NBUF]
    zsem = rest[5 * NBUF]
    c = lax.axis_index("c")
    s = lax.axis_index("s")
    wid = c * NS + s

    zero16 = jnp.zeros((16,), jnp.float32)
    one16 = jnp.ones((16,), jnp.float32)

    def fill_zrow(i, _):
        for j in range(D // 16):
            zrow[i, pl.ds(j * 16, 16)] = zero16
        return 0
    lax.fori_loop(0, ZR, fill_zrow, 0)

    def fill_czero(i, _):
        czero[pl.ds(i * 16, 16)] = zero16
        return 0
    lax.fori_loop(0, ROWS_PER_TILE // 16, fill_czero, 0)

    for off in sorted(set(list(range(0, K - 15, 16)) + [K - 16])):
        ones1[pl.ds(off, 16)] = one16

    r0 = s * ROWS_PER_TILE
    base = wid * EW
    d_cz = pltpu.async_copy(czero, cnt_s.at[pl.ds(r0, ROWS_PER_TILE)], zsem)

    def zero_agg(i, _):
        pltpu.async_copy(zrow, agg_s.at[pl.ds(r0 + i * ZR, ZR)], zsem)
        return 0
    lax.fori_loop(0, ROWS_PER_TILE // ZR, zero_agg, 0)

    def zero_agg_wait(i, _):
        pltpu.make_async_copy(zrow, agg_s.at[pl.ds(r0, ZR)], zsem).wait()
        return 0
    lax.fori_loop(0, ROWS_PER_TILE // ZR, zero_agg_wait, 0)
    d_cz.wait()

    plsc.subcore_barrier()

    def load_idx(j, b):
        pltpu.async_copy(src_hbm.at[pl.ds(base + j * K, K)], srcb[b], isem[b])
        pltpu.async_copy(dst_hbm.at[pl.ds(base + j * K, K)], didxb[b],
                         isem[b])

    def wait_idx(b):
        pltpu.make_async_copy(src_hbm.at[pl.ds(base, K)], srcb[b],
                              isem[b]).wait()
        pltpu.make_async_copy(dst_hbm.at[pl.ds(base, K)], didxb[b],
                              isem[b]).wait()

    def issue_gather(b):
        pltpu.async_copy(x_hbm.at[srcb[b]], rows.at[b], gsem[b])

    def wait_gather(b):
        pltpu.make_async_copy(x_hbm.at[srcb[b]], rows.at[b], gsem[b]).wait()

    def issue_scatter(b):
        pltpu.async_copy(rows.at[b], agg_s.at[didxb[b]], ssem[b], add=True)
        pltpu.async_copy(ones1, cnt_s.at[didxb[b]], ssem[b], add=True)

    def wait_scatter(b):
        pltpu.make_async_copy(rows.at[b], agg_s.at[didxb[b]], ssem[b]).wait()
        pltpu.make_async_copy(ones1, cnt_s.at[didxb[b]], ssem[b]).wait()

    for j in range(4):
        load_idx(j, j)
    for j in range(2):
        wait_idx(j)
        issue_gather(j)

    def outer(g, _):
        for b in range(NBUF):
            i = g * NBUF + b
            b2 = (b + 2) % NBUF
            b4 = (b + 4) % NBUF
            bp = (b - 1) % NBUF

            @pl.when(i + 4 < NCHUNK)
            def _load():
                load_idx(i + 4, b4)

            @pl.when(i + 2 < NCHUNK)
            def _gather():
                wait_idx(b2)
                issue_gather(b2)

            wait_gather(b)

            @pl.when(i >= 1)
            def _drain():
                wait_scatter(bp)

            issue_scatter(b)
        return 0
    lax.fori_loop(0, NCHUNK // NBUF, outer, 0)

    wait_scatter((NCHUNK - 1) % NBUF)

    plsc.subcore_barrier()

    pltpu.sync_copy(agg_s.at[pl.ds(r0, ROWS_PER_TILE)],
                    agg_out.at[c, pl.ds(r0, ROWS_PER_TILE)])
    pltpu.sync_copy(cnt_s.at[pl.ds(r0, ROWS_PER_TILE)],
                    cnt_out.at[pl.ds(c * NP + r0, ROWS_PER_TILE)])


@functools.lru_cache(maxsize=1)
def _sc_agg():
    return pl.kernel(
        _sc_body,
        mesh=plsc.VectorSubcoreMesh(core_axis_name="c", subcore_axis_name="s",
                                    num_cores=NC, num_subcores=NS),
        out_type=[jax.ShapeDtypeStruct((NC, NP, D), jnp.float32),
                  jax.ShapeDtypeStruct((NC * NP,), jnp.float32)],
        scratch_types=[
            pltpu.VMEM((ZR, D), jnp.float32),
            pltpu.VMEM((ROWS_PER_TILE,), jnp.float32),
            pltpu.VMEM((K,), jnp.float32),
            pltpu.VMEM((NBUF, K, D), jnp.float32),
            pltpu.VMEM_SHARED((NP, D), jnp.float32),
            pltpu.VMEM_SHARED((NP,), jnp.float32),
        ] + [pltpu.VMEM((K,), jnp.int32)] * (2 * NBUF)
          + [pltpu.SemaphoreType.DMA] * (3 * NBUF + 1),
    )


def _tc_body(agg_ref, cnt_ref, x_ref, wl_ref, bl_ref, wr_ref, wp_ref, bp_ref,
             ws_ref, bs_ref, outp_ref, outs_ref):
    p = agg_ref[0] + agg_ref[1]
    cnt = cnt_ref[0] + cnt_ref[1]
    inv = 1.0 / jnp.maximum(cnt, 1.0)
    aggm = p * inv
    h = jnp.dot(aggm, wl_ref[...], preferred_element_type=jnp.float32)
    h = h + bl_ref[...]
    h = h + jnp.dot(x_ref[...], wr_ref[...], preferred_element_type=jnp.float32)
    h = jnp.maximum(h, 0.0)

    def lsm(z):
        m = jnp.max(z, axis=1, keepdims=True)
        zz = z - m
        return zz - jnp.log(jnp.sum(jnp.exp(zz), axis=1, keepdims=True))

    lp = jnp.dot(h, wp_ref[...], preferred_element_type=jnp.float32)
    outp_ref[...] = lsm(lp + bp_ref[...])
    ls = jnp.dot(h, ws_ref[...], preferred_element_type=jnp.float32)
    outs_ref[...] = lsm(ls + bs_ref[...])


def _tc_head(agg_p, cnt_p, x, W_l, b_l, W_r, W_p, b_p, W_s, b_s):
    B = 2000
    grid = (N // B,)
    return pl.pallas_call(
        _tc_body,
        grid=grid,
        in_specs=[
            pl.BlockSpec((NC, B, D), lambda i: (0, i, 0)),
            pl.BlockSpec((NC, B, 1), lambda i: (0, i, 0)),
            pl.BlockSpec((B, D), lambda i: (i, 0)),
            pl.BlockSpec((D, H), lambda i: (0, 0)),
            pl.BlockSpec((1, H), lambda i: (0, 0)),
            pl.BlockSpec((D, H), lambda i: (0, 0)),
            pl.BlockSpec((H, 7), lambda i: (0, 0)),
            pl.BlockSpec((1, 7), lambda i: (0, 0)),
            pl.BlockSpec((H, 6), lambda i: (0, 0)),
            pl.BlockSpec((1, 6), lambda i: (0, 0)),
        ],
        out_specs=[pl.BlockSpec((B, 7), lambda i: (i, 0)),
                   pl.BlockSpec((B, 6), lambda i: (i, 0))],
        out_shape=[jax.ShapeDtypeStruct((N, 7), jnp.float32),
                   jax.ShapeDtypeStruct((N, 6), jnp.float32)],
    )(agg_p, cnt_p, x, W_l, b_l, W_r, W_p, b_p, W_s, b_s)


def kernel(x, edge_index, W_l, b_l, W_r, W_p, b_p, W_s, b_s):
    src = edge_index[0]
    dst = edge_index[1]
    agg_p, cnt_p = _sc_agg()(x, src, dst)
    cnt_p = cnt_p.reshape(NC, NP, 1)
    outp, outs = _tc_head(agg_p, cnt_p, x, W_l, b_l.reshape(1, H), W_r,
                          W_p, b_p.reshape(1, 7), W_s, b_s.reshape(1, 6))
    return (outp, outs)


# speedup vs baseline: 14.4528x; 1.0432x over previous
import functools

import jax
import jax.numpy as jnp
from jax import lax
from jax.experimental import pallas as pl
from jax.experimental.pallas import tpu as pltpu
from jax.experimental.pallas import tpu_sc as plsc

N = 10000
E = 320000
D = 128
H = 128

NC = 2
NS = 16
EW = E // (NC * NS)
K = 40
NCHUNK = EW // K
NP = 10240
ROWS_PER_TILE = NP // NS
ZR = 32
NBUF = 5
NW = NC * NS


def _sc_body(x_hbm, ei_hbm, agg_out, cnt_out,
             zrow, czero, ones1, rows, agg_s, cnt_s, *rest):
    srcb = rest[0:NBUF]
    didxb = rest[NBUF:2 * NBUF]
    isem = rest[2 * NBUF:3 * NBUF]
    gsem = rest[3 * NBUF:4 * NBUF]
    ssem = rest[4 * NBUF:5 * NBUF]
    zsem = rest[5 * NBUF]
    c = lax.axis_index("c")
    s = lax.axis_index("s")
    wid = c * NS + s

    zero16 = jnp.zeros((16,), jnp.float32)
    one16 = jnp.ones((16,), jnp.float32)

    def fill_zrow(i, _):
        for j in range(D // 16):
            zrow[i, pl.ds(j * 16, 16)] = zero16
        return 0
    lax.fori_loop(0, ZR, fill_zrow, 0)

    def fill_czero(i, _):
        czero[pl.ds(i * 16, 16)] = zero16
        return 0
    lax.fori_loop(0, ROWS_PER_TILE // 16, fill_czero, 0)

    for off in sorted(set(list(range(0, K - 15, 16)) + [K - 16])):
        ones1[pl.ds(off, 16)] = one16

    r0 = s * ROWS_PER_TILE
    base = wid * EW
    d_cz = pltpu.async_copy(czero, cnt_s.at[pl.ds(r0, ROWS_PER_TILE)], zsem)

    def zero_agg(i, _):
        pltpu.async_copy(zrow, agg_s.at[pl.ds(r0 + i * ZR, ZR)], zsem)
        return 0
    lax.fori_loop(0, ROWS_PER_TILE // ZR, zero_agg, 0)

    def zero_agg_wait(i, _):
        pltpu.make_async_copy(zrow, agg_s.at[pl.ds(r0, ZR)], zsem).wait()
        return 0
    lax.fori_loop(0, ROWS_PER_TILE // ZR, zero_agg_wait, 0)
    d_cz.wait()

    plsc.subcore_barrier()

    def load_idx(j, b):
        pltpu.async_copy(ei_hbm.at[pl.ds(base + j * K, K)], srcb[b], isem[b])
        pltpu.async_copy(ei_hbm.at[pl.ds(E + base + j * K, K)], didxb[b],
                         isem[b])

    def wait_idx(b):
        pltpu.make_async_copy(ei_hbm.at[pl.ds(base, K)], srcb[b],
                              isem[b]).wait()
        pltpu.make_async_copy(ei_hbm.at[pl.ds(base, K)], didxb[b],
                              isem[b]).wait()

    def issue_gather(b):
        pltpu.async_copy(x_hbm.at[srcb[b]], rows.at[b], gsem[b])

    def wait_gather(b):
        pltpu.make_async_copy(x_hbm.at[srcb[b]], rows.at[b], gsem[b]).wait()

    def issue_scatter(b):
        pltpu.async_copy(rows.at[b], agg_s.at[didxb[b]], ssem[b], add=True)
        pltpu.async_copy(ones1, cnt_s.at[didxb[b]], ssem[b], add=True)

    def wait_scatter(b):
        pltpu.make_async_copy(rows.at[b], agg_s.at[didxb[b]], ssem[b]).wait()
        pltpu.make_async_copy(ones1, cnt_s.at[didxb[b]], ssem[b]).wait()

    for j in range(4):
        load_idx(j, j)
    for j in range(2):
        wait_idx(j)
        issue_gather(j)

    def outer(g, _):
        for b in range(NBUF):
            i = g * NBUF + b
            b2 = (b + 2) % NBUF
            b4 = (b + 4) % NBUF
            bp = (b - 1) % NBUF

            @pl.when(i + 4 < NCHUNK)
            def _load():
                load_idx(i + 4, b4)

            @pl.when(i + 2 < NCHUNK)
            def _gather():
                wait_idx(b2)
                issue_gather(b2)

            wait_gather(b)
            issue_scatter(b)

            @pl.when(i >= 1)
            def _drain():
                wait_scatter(bp)
        return 0
    lax.fori_loop(0, NCHUNK // NBUF, outer, 0)

    wait_scatter((NCHUNK - 1) % NBUF)

    plsc.subcore_barrier()

    pltpu.sync_copy(agg_s.at[pl.ds(r0, ROWS_PER_TILE)],
                    agg_out.at[c, pl.ds(r0, ROWS_PER_TILE)])
    pltpu.sync_copy(cnt_s.at[pl.ds(r0, ROWS_PER_TILE)],
                    cnt_out.at[pl.ds(c * NP + r0, ROWS_PER_TILE)])


@functools.lru_cache(maxsize=1)
def _sc_agg():
    return pl.kernel(
        _sc_body,
        mesh=plsc.VectorSubcoreMesh(core_axis_name="c", subcore_axis_name="s",
                                    num_cores=NC, num_subcores=NS),
        out_type=[jax.ShapeDtypeStruct((NC, NP, D), jnp.float32),
                  jax.ShapeDtypeStruct((NC * NP,), jnp.float32)],
        scratch_types=[
            pltpu.VMEM((ZR, D), jnp.float32),
            pltpu.VMEM((ROWS_PER_TILE,), jnp.float32),
            pltpu.VMEM((K,), jnp.float32),
            pltpu.VMEM((NBUF, K, D), jnp.float32),
            pltpu.VMEM_SHARED((NP, D), jnp.float32),
            pltpu.VMEM_SHARED((NP,), jnp.float32),
        ] + [pltpu.VMEM((K,), jnp.int32)] * (2 * NBUF)
          + [pltpu.SemaphoreType.DMA] * (3 * NBUF + 1),
    )


def _tc_body(agg_ref, cnt_ref, x_ref, wl_ref, bl_ref, wr_ref, wp_ref, bp_ref,
             ws_ref, bs_ref, outp_ref, outs_ref):
    p = agg_ref[0] + agg_ref[1]
    cnt = cnt_ref[0] + cnt_ref[1]
    inv = 1.0 / jnp.maximum(cnt, 1.0)
    aggm = p * inv
    h = jnp.dot(aggm, wl_ref[...], preferred_element_type=jnp.float32)
    h = h + bl_ref[...]
    h = h + jnp.dot(x_ref[...], wr_ref[...], preferred_element_type=jnp.float32)
    h = jnp.maximum(h, 0.0)

    def lsm(z):
        m = jnp.max(z, axis=1, keepdims=True)
        zz = z - m
        return zz - jnp.log(jnp.sum(jnp.exp(zz), axis=1, keepdims=True))

    lp = jnp.dot(h, wp_ref[...], preferred_element_type=jnp.float32)
    outp_ref[...] = lsm(lp + bp_ref[...])
    ls = jnp.dot(h, ws_ref[...], preferred_element_type=jnp.float32)
    outs_ref[...] = lsm(ls + bs_ref[...])


def _tc_head(agg_p, cnt_p, x, W_l, b_l, W_r, W_p, b_p, W_s, b_s):
    B = 2000
    grid = (N // B,)
    return pl.pallas_call(
        _tc_body,
        grid=grid,
        in_specs=[
            pl.BlockSpec((NC, B, D), lambda i: (0, i, 0)),
            pl.BlockSpec((NC, B, 1), lambda i: (0, i, 0)),
            pl.BlockSpec((B, D), lambda i: (i, 0)),
            pl.BlockSpec((D, H), lambda i: (0, 0)),
            pl.BlockSpec((1, H), lambda i: (0, 0)),
            pl.BlockSpec((D, H), lambda i: (0, 0)),
            pl.BlockSpec((H, 7), lambda i: (0, 0)),
            pl.BlockSpec((1, 7), lambda i: (0, 0)),
            pl.BlockSpec((H, 6), lambda i: (0, 0)),
            pl.BlockSpec((1, 6), lambda i: (0, 0)),
        ],
        out_specs=[pl.BlockSpec((B, 7), lambda i: (i, 0)),
                   pl.BlockSpec((B, 6), lambda i: (i, 0))],
        out_shape=[jax.ShapeDtypeStruct((N, 7), jnp.float32),
                   jax.ShapeDtypeStruct((N, 6), jnp.float32)],
    )(agg_p, cnt_p, x, W_l, b_l, W_r, W_p, b_p, W_s, b_s)


def kernel(x, edge_index, W_l, b_l, W_r, W_p, b_p, W_s, b_s):
    agg_p, cnt_p = _sc_agg()(x, edge_index.reshape(2 * E))
    cnt_p = cnt_p.reshape(NC, NP, 1)
    outp, outs = _tc_head(agg_p, cnt_p, x, W_l, b_l.reshape(1, H), W_r,
                          W_p, b_p.reshape(1, 7), W_s, b_s.reshape(1, 6))
    return (outp, outs)
